# baseline (speedup 1.0000x reference)
import functools

import jax
import jax.numpy as jnp
from jax import lax
from jax.experimental import pallas as pl
from jax.experimental.pallas import tpu as pltpu
from jax.experimental.pallas import tpu_sc as plsc

D = 1024

NC, NS = 2, 16
NW = NC * NS

CHUNK = 64


def _embed_gather(idx_flat, tok_table):
    bt = idx_flat.shape[0]
    rows_per_w = bt // NW
    n_chunks = rows_per_w // CHUNK
    mesh = plsc.VectorSubcoreMesh(core_axis_name="c", subcore_axis_name="s")

    @functools.partial(
        pl.kernel,
        out_type=jax.ShapeDtypeStruct((bt, D), jnp.float32),
        mesh=mesh,
        scratch_types=[
            pltpu.VMEM((CHUNK,), jnp.int32),
            pltpu.VMEM((CHUNK, D), jnp.float32),
            pltpu.SemaphoreType.DMA,
        ],
    )
    def k(idx_hbm, tok_hbm, x_hbm, idx_v, rows_v, sem):
        wid = lax.axis_index("s") * NC + lax.axis_index("c")
        base = wid * rows_per_w
        for c in range(n_chunks):
            off = base + c * CHUNK
            pltpu.sync_copy(idx_hbm.at[pl.ds(off, CHUNK)], idx_v)
            pltpu.async_copy(tok_hbm.at[idx_v], rows_v, sem).wait()
            pltpu.sync_copy(rows_v, x_hbm.at[pl.ds(off, CHUNK)])

    return k(idx_flat, tok_table)


def _mm_body(x_ref, pos_ref, w_ref, b_ref, o_ref):
    xb = (x_ref[...] + pos_ref[...]).astype(jnp.bfloat16)
    acc = lax.dot_general(
        xb, w_ref[...], (((1,), (1,)), ((), ())),
        preferred_element_type=jnp.float32,
    )
    o_ref[...] = acc + b_ref[...]


def _matmul(x, pos_table, w_bf16, b2, t_len):
    bt = x.shape[0]
    v = w_bf16.shape[0]
    tm, tn = 1024, 1024
    t_tiles = t_len // tm
    return pl.pallas_call(
        _mm_body,
        grid=(bt // tm, v // tn),
        in_specs=[
            pl.BlockSpec((tm, D), lambda m, n: (m, 0)),
            pl.BlockSpec((tm, D), lambda m, n: (m % t_tiles, 0)),
            pl.BlockSpec((tn, D), lambda m, n: (n, 0)),
            pl.BlockSpec((1, tn), lambda m, n: (0, n)),
        ],
        out_specs=pl.BlockSpec((tm, tn), lambda m, n: (m, n)),
        out_shape=jax.ShapeDtypeStruct((bt, v), jnp.float32),
    )(x, pos_table, w_bf16, b2)


def kernel(idx, tok_table, pos_table, W, b):
    B, T = idx.shape
    v = W.shape[0]
    idx_flat = idx.reshape(-1).astype(jnp.int32)
    x = _embed_gather(idx_flat, tok_table)
    w_bf16 = W.astype(jnp.bfloat16)
    logits = _matmul(x, pos_table, w_bf16, b.reshape(1, -1), T)
    return logits.reshape(B, T, v)


# speedup vs baseline: 1.1197x; 1.1197x over previous
import functools

import jax
import jax.numpy as jnp
from jax import lax
from jax.experimental import pallas as pl
from jax.experimental.pallas import tpu as pltpu
from jax.experimental.pallas import tpu_sc as plsc

D = 1024

NC, NS = 2, 16
NW = NC * NS

CHUNK = 64

TM = 1024
TN = 1024


def _embed_gather(idx_flat, tok_table):
    bt = idx_flat.shape[0]
    rows_per_w = bt // NW
    n_chunks = rows_per_w // CHUNK
    mesh = plsc.VectorSubcoreMesh(core_axis_name="c", subcore_axis_name="s")

    @functools.partial(
        pl.kernel,
        out_type=jax.ShapeDtypeStruct((bt, D), jnp.float32),
        mesh=mesh,
        scratch_types=[
            pltpu.VMEM((CHUNK,), jnp.int32),
            pltpu.VMEM((CHUNK, D), jnp.float32),
            pltpu.SemaphoreType.DMA,
        ],
    )
    def k(idx_hbm, tok_hbm, x_hbm, idx_v, rows_v, sem):
        wid = lax.axis_index("s") * NC + lax.axis_index("c")
        base = wid * rows_per_w
        for c in range(n_chunks):
            off = base + c * CHUNK
            pltpu.sync_copy(idx_hbm.at[pl.ds(off, CHUNK)], idx_v)
            pltpu.async_copy(tok_hbm.at[idx_v], rows_v, sem).wait()
            pltpu.sync_copy(rows_v, x_hbm.at[pl.ds(off, CHUNK)])

    return k(idx_flat, tok_table)


def _mm_body(x_ref, pos_ref, w_ref, b_ref, o_ref):
    m = pl.program_id(1)
    t_tiles = pos_ref.shape[0] // TM
    toff = (m % t_tiles) * TM
    xs = (x_ref[pl.ds(m * TM, TM), :]
          + pos_ref[pl.ds(toff, TM), :]).astype(jnp.bfloat16)
    wb = w_ref[...].astype(jnp.bfloat16)
    acc = lax.dot_general(
        xs, wb, (((1,), (1,)), ((), ())),
        preferred_element_type=jnp.float32,
    )
    o_ref[...] = acc + b_ref[...]


def _matmul(x, pos_table, W, b2):
    bt = x.shape[0]
    v = W.shape[0]
    t_len = pos_table.shape[0]
    return pl.pallas_call(
        _mm_body,
        grid=(v // TN, bt // TM),
        in_specs=[
            pl.BlockSpec((bt, D), lambda n, m: (0, 0)),
            pl.BlockSpec((t_len, D), lambda n, m: (0, 0)),
            pl.BlockSpec((TN, D), lambda n, m: (n, 0)),
            pl.BlockSpec((1, TN), lambda n, m: (0, n)),
        ],
        out_specs=pl.BlockSpec((TM, TN), lambda n, m: (m, n)),
        out_shape=jax.ShapeDtypeStruct((bt, v), jnp.float32),
    )(x, pos_table, W, b2)


def kernel(idx, tok_table, pos_table, W, b):
    B, T = idx.shape
    v = W.shape[0]
    idx_flat = idx.reshape(-1).astype(jnp.int32)
    x = _embed_gather(idx_flat, tok_table)
    logits = _matmul(x, pos_table, W, b.reshape(1, -1))
    return logits.reshape(B, T, v)
